# baseline (speedup 1.0000x reference)
import functools
import math

import jax
import jax.numpy as jnp
from jax.experimental import pallas as pl
from jax.experimental.pallas import tpu as pltpu

B, N, VFEAT, QFEAT, MID, K_TOP, NK, ANS = 64, 100, 2048, 1024, 1024, 16, 8, 3129
VIN = VFEAT + 4
G = 8


def _dot(a, b):
    return jax.lax.dot_general(a, b, (((1,), (0,)), ((), ())),
                               preferred_element_type=jnp.float32)


def _fused_kernel(v_ref, b_ref, q_ref, cen_ref, cent_ref,
                  w1v_ref, w1b_ref, w1q_ref, b1_ref, w2_ref, b2_ref,
                  g1_ref, g2_ref, c1v_ref, c1b_ref, c2_ref, o_ref):
    logits_l = []
    for g in range(G):
        vc = v_ref[g]
        bx = b_ref[g]
        qp = _dot(q_ref[g], w1q_ref[...])
        h1 = _dot(vc, w1v_ref[...]) + _dot(bx, w1b_ref[...])
        h1 = jnp.maximum(h1 + qp + b1_ref[...][None, :], 0.0)
        h = jnp.maximum(_dot(h1, w2_ref[...]) + b2_ref[...][None, :], 0.0)
        logits_l.append(jax.lax.dot_general(
            h, h, (((1,), (1,)), ((), ())),
            preferred_element_type=jnp.float32))
    logits = jnp.stack(logits_l)

    col = jax.lax.broadcasted_iota(jnp.int32, (G, N, N), 2)
    rem = logits
    rowmax = None
    t = None
    for _ in range(K_TOP):
        cur = jnp.max(rem, axis=2, keepdims=True)
        if rowmax is None:
            rowmax = cur
        t = cur
        eq = rem == cur
        first = jnp.min(jnp.where(eq, col, jnp.int32(1 << 30)), axis=2,
                        keepdims=True)
        rem = jnp.where(col == first, jnp.float32(-3e38), rem)

    gt = logits > t
    count_gt = jnp.sum(gt.astype(jnp.float32), axis=2, keepdims=True)
    eqmask = logits == t
    lt_mat = (jax.lax.broadcasted_iota(jnp.int32, (N, N), 0) <
              jax.lax.broadcasted_iota(jnp.int32, (N, N), 1)).astype(jnp.float32)
    rank = jnp.stack([_dot(eqmask[g].astype(jnp.float32), lt_mat)
                      for g in range(G)])
    sel = gt | (eqmask & (rank < (K_TOP - count_gt)))

    e = jnp.where(sel, jnp.exp(logits - rowmax), 0.0)
    m_adj = e / jnp.sum(e, axis=2, keepdims=True)

    cen = cen_ref[...]
    cent = cent_ref[...]
    dx = cen[:, :, 0:1] - cent[:, 0:1, :]
    dy = cen[:, :, 1:2] - cent[:, 1:2, :]
    diag = jax.lax.broadcasted_iota(jnp.int32, (G, N, N), 1) == col
    rho = jnp.where(diag, 0.0, jnp.sqrt(dx * dx + dy * dy))
    theta = jnp.where(diag, 0.0, jnp.arctan2(dx, dy))

    two_pi = jnp.float32(2.0 * math.pi)

    def gauss_planes(g_ref):
        ws = []
        total = jnp.zeros((G, N, N), jnp.float32)
        for f in range(NK):
            d_r = rho - g_ref[0, f, 0]
            fa = jnp.abs(theta - g_ref[1, f, 0])
            wd = jnp.minimum(fa, jnp.abs(two_pi - fa))
            w = jnp.exp(-(d_r * d_r) * g_ref[2, f, 0]
                        - (wd * wd) * g_ref[3, f, 0])
            ws.append(w)
            total = total + w
        scale = m_adj / (total + 1e-14)
        return [w * scale for w in ws]

    a1 = gauss_planes(g1_ref)
    v16 = [v_ref[g].astype(jnp.bfloat16) for g in range(G)]
    b16 = [b_ref[g].astype(jnp.bfloat16) for g in range(G)]
    parts = [[None] * NK for _ in range(G)]
    for f in range(NK):
        wv, wb = c1v_ref[f], c1b_ref[f]
        for g in range(G):
            pf = _dot(v16[g], wv) + _dot(b16[g], wb)
            parts[g][f] = jnp.maximum(_dot(a1[f][g], pf), 0.0)
    x1_l = [jnp.concatenate(parts[g], axis=1).astype(jnp.bfloat16)
            for g in range(G)]

    a2 = gauss_planes(g2_ref)
    for f in range(NK):
        wf = c2_ref[f]
        for g in range(G):
            qf = _dot(x1_l[g], wf)
            x2 = jnp.maximum(_dot(a2[f][g], qf), 0.0)
            o_ref[g, 0, f * (MID // NK):(f + 1) * (MID // NK)] = (
                jnp.max(x2, axis=0))


def _cls_kernel(pooled_ref, q_ref, w1_ref, b1_ref, w2_ref, b2_ref, o_ref):
    x = pooled_ref[...] * jnp.maximum(q_ref[...], 0.0)
    h = jnp.maximum(_dot(x, w1_ref[...]) + b1_ref[...][None, :], 0.0)
    o_ref[...] = _dot(h, w2_ref[...]) + b2_ref[...][None, :]


def kernel(v, b, q_feat, v_mask, gl_w1, gl_b1, gl_w2, gl_b2,
           gc1_w, gc1_mr, gc1_mt, gc1_pr, gc1_pt,
           gc2_w, gc2_mr, gc2_mt, gc2_pr, gc2_pt,
           cls_w1, cls_b1, cls_w2, cls_b2):
    f32 = jnp.float32
    centers = (b[:, :, 2:] + b[:, :, :2]) * 0.5
    centers_t = jnp.transpose(centers, (0, 2, 1))
    w1v, w1b, w1q = gl_w1[:VFEAT], gl_w1[VFEAT:VIN], gl_w1[VIN:]
    c1v = gc1_w[:, :VFEAT].astype(jnp.bfloat16)
    c1b = gc1_w[:, VFEAT:].astype(jnp.bfloat16)
    c2_16 = gc2_w.astype(jnp.bfloat16)

    def pack(mr, mt, pr, pt):
        return jnp.stack([mr, mt, 0.5 / (1e-14 + pr * pr),
                          0.5 / (1e-14 + pt * pt)]).reshape(4, NK, 1)

    g1 = pack(gc1_mr, gc1_mt, gc1_pr, gc1_pt)
    g2 = pack(gc2_mr, gc2_mt, gc2_pr, gc2_pt)

    rep = lambda *blk: pl.BlockSpec(blk, lambda i: (0,) * len(blk))
    per_b = lambda *blk: pl.BlockSpec(blk, lambda i: (i,) + (0,) * (len(blk) - 1))

    pooled = pl.pallas_call(
        _fused_kernel,
        grid=(B // G,),
        in_specs=[per_b(G, N, VFEAT), per_b(G, N, 4), per_b(G, 1, QFEAT),
                  per_b(G, N, 2), per_b(G, 2, N),
                  rep(VFEAT, 512), rep(4, 512), rep(QFEAT, 512), rep(512),
                  rep(512, 512), rep(512), rep(4, NK, 1), rep(4, NK, 1),
                  rep(NK, VFEAT, (2 * MID) // NK), rep(NK, 4, (2 * MID) // NK),
                  rep(NK, 2 * MID, MID // NK)],
        out_specs=per_b(G, 1, MID),
        out_shape=jax.ShapeDtypeStruct((B, 1, MID), f32),
        compiler_params=pltpu.CompilerParams(
            vmem_limit_bytes=100 * 1024 * 1024),
    )(v, b, q_feat.reshape(B, 1, QFEAT), centers, centers_t,
      w1v, w1b, w1q, gl_b1, gl_w2, gl_b2, g1, g2, c1v, c1b, c2_16
      ).reshape(B, MID)

    out = pl.pallas_call(
        _cls_kernel,
        in_specs=[pl.BlockSpec((B, MID), lambda: (0, 0)),
                  pl.BlockSpec((B, QFEAT), lambda: (0, 0)),
                  pl.BlockSpec((MID, 2 * MID), lambda: (0, 0)),
                  pl.BlockSpec((2 * MID,), lambda: (0,)),
                  pl.BlockSpec((2 * MID, ANS), lambda: (0, 0)),
                  pl.BlockSpec((ANS,), lambda: (0,))],
        out_specs=pl.BlockSpec((B, ANS), lambda: (0, 0)),
        out_shape=jax.ShapeDtypeStruct((B, ANS), f32),
    )(pooled, q_feat, cls_w1, cls_b1, cls_w2, cls_b2)
    return out


# speedup vs baseline: 1.7737x; 1.7737x over previous
import functools
import math

import jax
import jax.numpy as jnp
from jax.experimental import pallas as pl
from jax.experimental.pallas import tpu as pltpu

B, N, VFEAT, QFEAT, MID, K_TOP, NK, ANS = 64, 100, 2048, 1024, 1024, 16, 8, 3129
VIN = VFEAT + 4
G = 1


def _dot(a, b):
    return jax.lax.dot_general(a, b, (((1,), (0,)), ((), ())),
                               preferred_element_type=jnp.float32)


def _fused_kernel(v_ref, b_ref, q_ref, cen_ref, cent_ref,
                  w1v_ref, w1b_ref, w1q_ref, b1_ref, w2_ref, b2_ref,
                  g1_ref, g2_ref, c1v_ref, c1b_ref, c2_ref, o_ref):
    logits_l = []
    for g in range(G):
        vc = v_ref[g]
        bx = b_ref[g]
        qp = _dot(q_ref[g], w1q_ref[...])
        h1 = _dot(vc, w1v_ref[...]) + _dot(bx, w1b_ref[...])
        h1 = jnp.maximum(h1 + qp + b1_ref[...][None, :], 0.0)
        h = jnp.maximum(_dot(h1, w2_ref[...]) + b2_ref[...][None, :], 0.0)
        logits_l.append(jax.lax.dot_general(
            h, h, (((1,), (1,)), ((), ())),
            preferred_element_type=jnp.float32))
    logits = jnp.stack(logits_l)

    col = jax.lax.broadcasted_iota(jnp.int32, (G, N, N), 2)
    cen = cen_ref[...]
    cent = cent_ref[...]
    dx = cen[:, :, 0:1] - cent[:, 0:1, :]
    dy = cen[:, :, 1:2] - cent[:, 1:2, :]
    diag = jax.lax.broadcasted_iota(jnp.int32, (G, N, N), 1) == col
    rho = jnp.where(diag, 0.0, jnp.sqrt(dx * dx + dy * dy))
    theta = jnp.where(diag, 0.0, jnp.arctan2(dx, dy))
    two_pi = jnp.float32(2.0 * math.pi)

    def gauss_raw(g_ref):
        ws = []
        total = jnp.zeros((G, N, N), jnp.float32)
        for f in range(NK):
            d_r = rho - g_ref[0, f, 0]
            fa = jnp.abs(theta - g_ref[1, f, 0])
            wd = jnp.minimum(fa, jnp.abs(two_pi - fa))
            w = jnp.exp(-(d_r * d_r) * g_ref[2, f, 0]
                        - (wd * wd) * g_ref[3, f, 0])
            ws.append(w)
            total = total + w
        return ws, total

    ws1, tot1 = gauss_raw(g1_ref)
    ws2, tot2 = gauss_raw(g2_ref)

    v16 = [v_ref[g].astype(jnp.bfloat16) for g in range(G)]
    b16 = [b_ref[g].astype(jnp.bfloat16) for g in range(G)]
    pfs = [[None] * NK for _ in range(G)]
    for f in range(NK):
        wv, wb = c1v_ref[f], c1b_ref[f]
        for g in range(G):
            pfs[g][f] = _dot(v16[g], wv) + _dot(b16[g], wb)

    rem = logits
    rowmax = None
    t = None
    cum = jnp.zeros((G, N, 1), jnp.float32)
    for _ in range(K_TOP):
        cur = jnp.max(rem, axis=2, keepdims=True)
        if rowmax is None:
            rowmax = cur
            t = cur
        else:
            t = jnp.where(cum < K_TOP, cur, t)
        eq = rem == cur
        cum = cum + jnp.sum(eq.astype(jnp.float32), axis=2, keepdims=True)
        rem = jnp.where(eq, jnp.float32(-3e38), rem)

    gt = logits > t
    count_gt = jnp.sum(gt.astype(jnp.float32), axis=2, keepdims=True)
    eqmask = logits == t
    lt_mat = (jax.lax.broadcasted_iota(jnp.int32, (N, N), 0) <
              jax.lax.broadcasted_iota(jnp.int32, (N, N), 1)).astype(jnp.float32)
    rank = jnp.stack([_dot(eqmask[g].astype(jnp.float32), lt_mat)
                      for g in range(G)])
    sel = gt | (eqmask & (rank < (K_TOP - count_gt)))

    e = jnp.where(sel, jnp.exp(logits - rowmax), 0.0)
    m_adj = e / jnp.sum(e, axis=2, keepdims=True)

    scale1 = m_adj / (tot1 + 1e-14)
    a1 = [w * scale1 for w in ws1]
    parts = [[None] * NK for _ in range(G)]
    for f in range(NK):
        for g in range(G):
            parts[g][f] = jnp.maximum(_dot(a1[f][g], pfs[g][f]), 0.0)
    x1_l = [jnp.concatenate(parts[g], axis=1).astype(jnp.bfloat16)
            for g in range(G)]

    qfs = [[None] * NK for _ in range(G)]
    for f in range(NK):
        wf = c2_ref[f]
        for g in range(G):
            qfs[g][f] = _dot(x1_l[g], wf)
    scale2 = m_adj / (tot2 + 1e-14)
    a2 = [w * scale2 for w in ws2]
    for f in range(NK):
        for g in range(G):
            x2 = jnp.maximum(_dot(a2[f][g], qfs[g][f]), 0.0)
            o_ref[g, 0, f * (MID // NK):(f + 1) * (MID // NK)] = (
                jnp.max(x2, axis=0))


def _cls_kernel(pooled_ref, q_ref, w1_ref, b1_ref, w2_ref, b2_ref, o_ref):
    x = pooled_ref[...] * jnp.maximum(q_ref[...], 0.0)
    h = jnp.maximum(_dot(x, w1_ref[...]) + b1_ref[...][None, :], 0.0)
    o_ref[...] = _dot(h, w2_ref[...]) + b2_ref[...][None, :]


def kernel(v, b, q_feat, v_mask, gl_w1, gl_b1, gl_w2, gl_b2,
           gc1_w, gc1_mr, gc1_mt, gc1_pr, gc1_pt,
           gc2_w, gc2_mr, gc2_mt, gc2_pr, gc2_pt,
           cls_w1, cls_b1, cls_w2, cls_b2):
    f32 = jnp.float32
    centers = (b[:, :, 2:] + b[:, :, :2]) * 0.5
    centers_t = jnp.transpose(centers, (0, 2, 1))
    w1v, w1b, w1q = gl_w1[:VFEAT], gl_w1[VFEAT:VIN], gl_w1[VIN:]
    c1v = gc1_w[:, :VFEAT].astype(jnp.bfloat16)
    c1b = gc1_w[:, VFEAT:].astype(jnp.bfloat16)
    c2_16 = gc2_w.astype(jnp.bfloat16)

    def pack(mr, mt, pr, pt):
        return jnp.stack([mr, mt, 0.5 / (1e-14 + pr * pr),
                          0.5 / (1e-14 + pt * pt)]).reshape(4, NK, 1)

    g1 = pack(gc1_mr, gc1_mt, gc1_pr, gc1_pt)
    g2 = pack(gc2_mr, gc2_mt, gc2_pr, gc2_pt)

    rep = lambda *blk: pl.BlockSpec(blk, lambda i: (0,) * len(blk))
    per_b = lambda *blk: pl.BlockSpec(blk, lambda i: (i,) + (0,) * (len(blk) - 1))

    pooled = pl.pallas_call(
        _fused_kernel,
        grid=(B // G,),
        in_specs=[per_b(G, N, VFEAT), per_b(G, N, 4), per_b(G, 1, QFEAT),
                  per_b(G, N, 2), per_b(G, 2, N),
                  rep(VFEAT, 512), rep(4, 512), rep(QFEAT, 512), rep(512),
                  rep(512, 512), rep(512), rep(4, NK, 1), rep(4, NK, 1),
                  rep(NK, VFEAT, (2 * MID) // NK), rep(NK, 4, (2 * MID) // NK),
                  rep(NK, 2 * MID, MID // NK)],
        out_specs=per_b(G, 1, MID),
        out_shape=jax.ShapeDtypeStruct((B, 1, MID), f32),
        compiler_params=pltpu.CompilerParams(
            vmem_limit_bytes=100 * 1024 * 1024),
    )(v, b, q_feat.reshape(B, 1, QFEAT), centers, centers_t,
      w1v, w1b, w1q, gl_b1, gl_w2, gl_b2, g1, g2, c1v, c1b, c2_16
      ).reshape(B, MID)

    out = pl.pallas_call(
        _cls_kernel,
        in_specs=[pl.BlockSpec((B, MID), lambda: (0, 0)),
                  pl.BlockSpec((B, QFEAT), lambda: (0, 0)),
                  pl.BlockSpec((MID, 2 * MID), lambda: (0, 0)),
                  pl.BlockSpec((2 * MID,), lambda: (0,)),
                  pl.BlockSpec((2 * MID, ANS), lambda: (0, 0)),
                  pl.BlockSpec((ANS,), lambda: (0,))],
        out_specs=pl.BlockSpec((B, ANS), lambda: (0, 0)),
        out_shape=jax.ShapeDtypeStruct((B, ANS), f32),
    )(pooled, q_feat, cls_w1, cls_b1, cls_w2, cls_b2)
    return out
